# baseline (speedup 1.0000x reference)
import functools

import jax
import jax.numpy as jnp
from jax import lax
from jax.experimental import pallas as pl
from jax.experimental.pallas import tpu as pltpu
from jax.experimental.pallas import tpu_sc as plsc

WORD_VOCAB = 100000
D = 64
B = 4096
L = 200
NT = B * L
NC, NS = 2, 16
NW = NC * NS
ROWS_PER_W = B // NW
IBLK = 16
H0, H1 = 104, 96


_R = 4000


def _transform_body(vt_ref, pt_ref, w_ref, b_ref, vtt_ref, ptt_ref):
    w = w_ref[...]
    wv = w[:, 0:64]
    wp = w[:, 64:128]
    dn = (((1,), (1,)), ((), ()))
    vtt_ref[...] = (
        lax.dot_general(vt_ref[...], wv, dn, precision=lax.Precision.HIGHEST)
        + b_ref[...]
    )
    ptt_ref[...] = lax.dot_general(
        pt_ref[...], wp, dn, precision=lax.Precision.HIGHEST
    )


def _transform_tables(value_table, pos_table, W, b):
    b2 = b.reshape(1, D)
    return pl.pallas_call(
        _transform_body,
        grid=(WORD_VOCAB // _R,),
        in_specs=[
            pl.BlockSpec((_R, D), lambda i: (i, 0)),
            pl.BlockSpec((_R, D), lambda i: (i, 0)),
            pl.BlockSpec((D, 2 * D), lambda i: (0, 0)),
            pl.BlockSpec((1, D), lambda i: (0, 0)),
        ],
        out_specs=[
            pl.BlockSpec((_R, D), lambda i: (i, 0)),
            pl.BlockSpec((_R, D), lambda i: (i, 0)),
        ],
        out_shape=[
            jax.ShapeDtypeStruct((WORD_VOCAB, D), jnp.float32),
            jax.ShapeDtypeStruct((WORD_VOCAB, D), jnp.float32),
        ],
    )(value_table, pos_table, W, b2)


def _sc_body(vtt, ptt, pt, vi, pi, merged, pos,
             ivi, ipi, va0, vb0, vc0, va1, vb1, vc1,
             gs0, gs1, ss0, ss1):
    wid = lax.axis_index("s") * NC + lax.axis_index("c")
    base_row = wid * ROWS_PER_W

    def stage_idx(g):
        t0 = (base_row + g) * L
        pltpu.sync_copy(vi.at[pl.ds(t0, IBLK * L)], ivi)
        pltpu.sync_copy(pi.at[pl.ds(t0, IBLK * L)], ipi)

    def gather_copies(g, va, vb, vc, sem):
        jj = lax.rem(g, IBLK)
        out = []
        for off, width in ((0, H0), (H0, H1)):
            isl = pl.ds(jj * L + off, width)
            dst = pl.ds(off, width)
            out.append(pltpu.make_async_copy(
                vtt.at[ivi.at[isl]], va.at[dst], sem))
            out.append(pltpu.make_async_copy(
                ptt.at[ipi.at[isl]], vb.at[dst], sem))
            out.append(pltpu.make_async_copy(
                pt.at[ipi.at[isl]], vc.at[dst], sem))
        return out

    def issue_gathers(g, va, vb, vc, sem):
        for c in gather_copies(g, va, vb, vc, sem):
            c.start()

    def wait_gathers(g, va, vb, vc, sem):
        for c in gather_copies(g, va, vb, vc, sem):
            c.wait()

    def scatter_copies(g, va, vc, sem):
        t0 = (base_row + g) * L
        return [
            pltpu.make_async_copy(va, merged.at[pl.ds(t0, L)], sem),
            pltpu.make_async_copy(vc, pos.at[pl.ds(t0, L)], sem),
        ]

    def compute(va, vb):
        def tok(t, carry):
            for dd in range(D // 16):
                sl = pl.ds(dd * 16, 16)
                va[t, sl] = jnp.maximum(va[t, sl] + vb[t, sl], 0.0)
            return carry
        lax.fori_loop(0, L, tok, 0)

    def loop_body(i, carry):
        g0 = 2 * i
        g1 = 2 * i + 1
        @pl.when(lax.rem(g0 + 1, IBLK) == 0)
        def _():
            stage_idx(g0 + 1)
        @pl.when(i >= 1)
        def _():
            for c in scatter_copies(g0 - 1, va1, vc1, ss1):
                c.wait()
        issue_gathers(g0 + 1, va1, vb1, vc1, gs1)
        wait_gathers(g0, va0, vb0, vc0, gs0)
        compute(va0, vb0)
        for c in scatter_copies(g0, va0, vc0, ss0):
            c.start()
        @pl.when(i < (ROWS_PER_W // 2) - 1)
        def _():
            @pl.when(lax.rem(g1 + 1, IBLK) == 0)
            def _():
                stage_idx(g1 + 1)
            for c in scatter_copies(g0, va0, vc0, ss0):
                c.wait()
            issue_gathers(g1 + 1, va0, vb0, vc0, gs0)
        wait_gathers(g1, va1, vb1, vc1, gs1)
        compute(va1, vb1)
        for c in scatter_copies(g1, va1, vc1, ss1):
            c.start()
        return carry

    stage_idx(0)
    issue_gathers(0, va0, vb0, vc0, gs0)
    lax.fori_loop(0, ROWS_PER_W // 2, loop_body, 0)
    for c in scatter_copies(ROWS_PER_W - 2, va0, vc0, ss0):
        c.wait()
    for c in scatter_copies(ROWS_PER_W - 1, va1, vc1, ss1):
        c.wait()


_sc_gather = functools.partial(
    pl.kernel,
    out_type=[
        jax.ShapeDtypeStruct((NT, D), jnp.float32),
        jax.ShapeDtypeStruct((NT, D), jnp.float32),
    ],
    mesh=plsc.VectorSubcoreMesh(core_axis_name="c", subcore_axis_name="s"),
    compiler_params=pltpu.CompilerParams(use_tc_tiling_on_sc=False),
    scratch_types=[
        pltpu.VMEM((IBLK * L,), jnp.int32),
        pltpu.VMEM((IBLK * L,), jnp.int32),
        pltpu.VMEM((L, D), jnp.float32),
        pltpu.VMEM((L, D), jnp.float32),
        pltpu.VMEM((L, D), jnp.float32),
        pltpu.VMEM((L, D), jnp.float32),
        pltpu.VMEM((L, D), jnp.float32),
        pltpu.VMEM((L, D), jnp.float32),
        pltpu.SemaphoreType.DMA,
        pltpu.SemaphoreType.DMA,
        pltpu.SemaphoreType.DMA,
        pltpu.SemaphoreType.DMA,
    ],
)(_sc_body)


_BB = 16


def _relayout_body(m2_ref, p2_ref, m3_ref, p3_ref):
    m3_ref[...] = m2_ref[...].reshape(_BB, L, D)
    p3_ref[...] = p2_ref[...].reshape(_BB, L, D)


def _relayout(m2, p2):
    return pl.pallas_call(
        _relayout_body,
        grid=(B // _BB,),
        in_specs=[
            pl.BlockSpec((_BB * L, D), lambda i: (i, 0)),
            pl.BlockSpec((_BB * L, D), lambda i: (i, 0)),
        ],
        out_specs=[
            pl.BlockSpec((_BB, L, D), lambda i: (i, 0, 0)),
            pl.BlockSpec((_BB, L, D), lambda i: (i, 0, 0)),
        ],
        out_shape=[
            jax.ShapeDtypeStruct((B, L, D), jnp.float32),
            jax.ShapeDtypeStruct((B, L, D), jnp.float32),
        ],
    )(m2, p2)


def kernel(value_table, pos_table, W, b, inputs):
    vtt, ptt = _transform_tables(value_table, pos_table, W, b)
    flat = inputs.reshape(NT, 2)
    vi = flat[:, 0]
    pi = flat[:, 1]
    m2, p2 = _sc_gather(vtt, ptt, pos_table, vi, pi)
    return _relayout(m2, p2)


# speedup vs baseline: 1.3811x; 1.3811x over previous
import functools

import jax
import jax.numpy as jnp
from jax import lax
from jax.experimental import pallas as pl
from jax.experimental.pallas import tpu as pltpu
from jax.experimental.pallas import tpu_sc as plsc

WORD_VOCAB = 100000
D = 64
B = 4096
L = 200
NC, NS = 2, 16
NW = NC * NS
BW = B // NW
IBLK = 16


_R = 4000


def _transform_body(vt_ref, pt_ref, w_ref, b_ref, vtt_ref, ptt_ref):
    w = w_ref[...]
    wv = w[:, 0:64]
    wp = w[:, 64:128]
    dn = (((1,), (1,)), ((), ()))
    vtt_ref[...] = (
        lax.dot_general(vt_ref[...], wv, dn, precision=lax.Precision.HIGHEST)
        + b_ref[...]
    )
    ptt_ref[...] = lax.dot_general(
        pt_ref[...], wp, dn, precision=lax.Precision.HIGHEST
    )


def _transform_tables(value_table, pos_table, W, b):
    b2 = b.reshape(1, D)
    return pl.pallas_call(
        _transform_body,
        grid=(WORD_VOCAB // _R,),
        in_specs=[
            pl.BlockSpec((_R, D), lambda i: (i, 0)),
            pl.BlockSpec((_R, D), lambda i: (i, 0)),
            pl.BlockSpec((D, 2 * D), lambda i: (0, 0)),
            pl.BlockSpec((1, D), lambda i: (0, 0)),
        ],
        out_specs=[
            pl.BlockSpec((_R, D), lambda i: (i, 0)),
            pl.BlockSpec((_R, D), lambda i: (i, 0)),
        ],
        out_shape=[
            jax.ShapeDtypeStruct((WORD_VOCAB, D), jnp.float32),
            jax.ShapeDtypeStruct((WORD_VOCAB, D), jnp.float32),
        ],
    )(value_table, pos_table, W, b2)


def _sc_body(vtt, ptt, pt, vi, pi, merged, pos,
             ivi, ipi, va0, vb0, vc0, va1, vb1, vc1,
             gs0, gs1, ss0, ss1):
    wid = lax.axis_index("s") * NC + lax.axis_index("c")
    b0 = wid * BW

    def stage_idx(g):
        pltpu.sync_copy(vi.at[pl.ds(g, IBLK), pl.ds(b0, BW)], ivi)
        pltpu.sync_copy(pi.at[pl.ds(g, IBLK), pl.ds(b0, BW)], ipi)

    def gather_copies(g, va, vb, vc, sem):
        jj = lax.rem(g, IBLK)
        return [
            pltpu.make_async_copy(vtt.at[ivi.at[jj]], va, sem),
            pltpu.make_async_copy(ptt.at[ipi.at[jj]], vb, sem),
            pltpu.make_async_copy(pt.at[ipi.at[jj]], vc, sem),
        ]

    def issue_gathers(g, va, vb, vc, sem):
        for c in gather_copies(g, va, vb, vc, sem):
            c.start()

    def wait_gathers(g, va, vb, vc, sem):
        for c in gather_copies(g, va, vb, vc, sem):
            c.wait()

    def scatter_copies(g, va, vc, sem):
        return [
            pltpu.make_async_copy(va, merged.at[g, pl.ds(b0, BW)], sem),
            pltpu.make_async_copy(vc, pos.at[g, pl.ds(b0, BW)], sem),
        ]

    def compute(va, vb):
        def tok(t, carry):
            for dd in range(D // 16):
                sl = pl.ds(dd * 16, 16)
                va[t, sl] = jnp.maximum(va[t, sl] + vb[t, sl], 0.0)
            return carry
        lax.fori_loop(0, BW, tok, 0)

    def loop_body(i, carry):
        g0 = 2 * i
        g1 = 2 * i + 1
        @pl.when(lax.rem(g0 + 1, IBLK) == 0)
        def _():
            stage_idx(g0 + 1)
        @pl.when(i >= 1)
        def _():
            for c in scatter_copies(g0 - 1, va1, vc1, ss1):
                c.wait()
        issue_gathers(g0 + 1, va1, vb1, vc1, gs1)
        wait_gathers(g0, va0, vb0, vc0, gs0)
        compute(va0, vb0)
        for c in scatter_copies(g0, va0, vc0, ss0):
            c.start()
        @pl.when(i < (L // 2) - 1)
        def _():
            @pl.when(lax.rem(g1 + 1, IBLK) == 0)
            def _():
                stage_idx(g1 + 1)
            for c in scatter_copies(g0, va0, vc0, ss0):
                c.wait()
            issue_gathers(g1 + 1, va0, vb0, vc0, gs0)
        wait_gathers(g1, va1, vb1, vc1, gs1)
        compute(va1, vb1)
        for c in scatter_copies(g1, va1, vc1, ss1):
            c.start()
        return carry

    stage_idx(0)
    issue_gathers(0, va0, vb0, vc0, gs0)
    lax.fori_loop(0, L // 2, loop_body, 0)
    for c in scatter_copies(L - 2, va0, vc0, ss0):
        c.wait()
    for c in scatter_copies(L - 1, va1, vc1, ss1):
        c.wait()


_sc_gather = functools.partial(
    pl.kernel,
    out_type=[
        jax.ShapeDtypeStruct((L, B, D), jnp.float32),
        jax.ShapeDtypeStruct((L, B, D), jnp.float32),
    ],
    mesh=plsc.VectorSubcoreMesh(core_axis_name="c", subcore_axis_name="s"),
    compiler_params=pltpu.CompilerParams(use_tc_tiling_on_sc=False),
    scratch_types=[
        pltpu.VMEM((IBLK, BW), jnp.int32),
        pltpu.VMEM((IBLK, BW), jnp.int32),
        pltpu.VMEM((BW, D), jnp.float32),
        pltpu.VMEM((BW, D), jnp.float32),
        pltpu.VMEM((BW, D), jnp.float32),
        pltpu.VMEM((BW, D), jnp.float32),
        pltpu.VMEM((BW, D), jnp.float32),
        pltpu.VMEM((BW, D), jnp.float32),
        pltpu.SemaphoreType.DMA,
        pltpu.SemaphoreType.DMA,
        pltpu.SemaphoreType.DMA,
        pltpu.SemaphoreType.DMA,
    ],
)(_sc_body)


_DT, _DR = D // 8, 8
_BT, _BC = B // 128, 128


def _relayout_body(m2_ref, p2_ref, m5_ref, p5_ref):
    for src, dst in ((m2_ref, m5_ref), (p2_ref, p5_ref)):
        x = src[0]
        xt = x.T
        dst[0] = xt.reshape(_DT, _DR, _BT, _BC).transpose(0, 2, 1, 3)


def _relayout(m2, p2):
    m5, p5 = pl.pallas_call(
        _relayout_body,
        grid=(L,),
        in_specs=[
            pl.BlockSpec((1, B, D), lambda i: (i, 0, 0)),
            pl.BlockSpec((1, B, D), lambda i: (i, 0, 0)),
        ],
        out_specs=[
            pl.BlockSpec((1, _DT, _BT, _DR, _BC), lambda i: (i, 0, 0, 0, 0)),
            pl.BlockSpec((1, _DT, _BT, _DR, _BC), lambda i: (i, 0, 0, 0, 0)),
        ],
        out_shape=[
            jax.ShapeDtypeStruct((L, _DT, _BT, _DR, _BC), jnp.float32),
            jax.ShapeDtypeStruct((L, _DT, _BT, _DR, _BC), jnp.float32),
        ],
    )(m2, p2)

    def to3d(x5):
        return x5.transpose(2, 4, 0, 1, 3).reshape(B, L, D)

    return to3d(m5), to3d(p5)


def kernel(value_table, pos_table, W, b, inputs):
    vtt, ptt = _transform_tables(value_table, pos_table, W, b)
    vi = inputs[:, :, 0].T
    pi = inputs[:, :, 1].T
    m2, p2 = _sc_gather(vtt, ptt, pos_table, vi, pi)
    return _relayout(m2, p2)


# speedup vs baseline: 2.2150x; 1.6037x over previous
import functools

import jax
import jax.numpy as jnp
from jax import lax
from jax.experimental import pallas as pl
from jax.experimental.pallas import tpu as pltpu
from jax.experimental.pallas import tpu_sc as plsc

WORD_VOCAB = 100000
D = 64
B = 4096
L = 200
NC, NS = 2, 16
NW = NC * NS
BW = B // NW
IBLK = 16


_R = 4000


def _transform_body(vt_ref, pt_ref, w_ref, b_ref, vtt_ref, ptt_ref):
    w = w_ref[...]
    wv = w[:, 0:64]
    wp = w[:, 64:128]
    dn = (((1,), (1,)), ((), ()))
    vtt_ref[...] = (
        lax.dot_general(vt_ref[...], wv, dn, precision=lax.Precision.HIGHEST)
        + b_ref[...]
    )
    ptt_ref[...] = lax.dot_general(
        pt_ref[...], wp, dn, precision=lax.Precision.HIGHEST
    )


def _transform_tables(value_table, pos_table, W, b):
    b2 = b.reshape(1, D)
    return pl.pallas_call(
        _transform_body,
        grid=(WORD_VOCAB // _R,),
        in_specs=[
            pl.BlockSpec((_R, D), lambda i: (i, 0)),
            pl.BlockSpec((_R, D), lambda i: (i, 0)),
            pl.BlockSpec((D, 2 * D), lambda i: (0, 0)),
            pl.BlockSpec((1, D), lambda i: (0, 0)),
        ],
        out_specs=[
            pl.BlockSpec((_R, D), lambda i: (i, 0)),
            pl.BlockSpec((_R, D), lambda i: (i, 0)),
        ],
        out_shape=[
            jax.ShapeDtypeStruct((WORD_VOCAB, D), jnp.float32),
            jax.ShapeDtypeStruct((WORD_VOCAB, D), jnp.float32),
        ],
    )(value_table, pos_table, W, b2)


def _sc_body(vtt, ptt, pt, vi, pi, mp,
             ivi, ipi, va0, vb0, vc0, va1, vb1, vc1,
             gs0, gs1, ss0, ss1):
    wid = lax.axis_index("s") * NC + lax.axis_index("c")
    b0 = wid * BW

    def stage_idx(g):
        pltpu.sync_copy(vi.at[pl.ds(g, IBLK), pl.ds(b0, BW)], ivi)
        pltpu.sync_copy(pi.at[pl.ds(g, IBLK), pl.ds(b0, BW)], ipi)

    def gather_copies(g, va, vb, vc, sem):
        jj = lax.rem(g, IBLK)
        return [
            pltpu.make_async_copy(vtt.at[ivi.at[jj]], va, sem),
            pltpu.make_async_copy(ptt.at[ipi.at[jj]], vb, sem),
            pltpu.make_async_copy(pt.at[ipi.at[jj]], vc, sem),
        ]

    def issue_gathers(g, va, vb, vc, sem):
        for c in gather_copies(g, va, vb, vc, sem):
            c.start()

    def wait_gathers(g, va, vb, vc, sem):
        for c in gather_copies(g, va, vb, vc, sem):
            c.wait()

    def scatter_copies(g, va, vc, sem):
        dst = mp.at[g, pl.ds(b0, BW)]
        return [
            pltpu.make_async_copy(va, dst.at[:, pl.ds(0, D)], sem),
            pltpu.make_async_copy(vc, dst.at[:, pl.ds(D, D)], sem),
        ]

    def compute(va, vb):
        def tok(t, carry):
            for dd in range(D // 16):
                sl = pl.ds(dd * 16, 16)
                va[t, sl] = jnp.maximum(va[t, sl] + vb[t, sl], 0.0)
            return carry
        lax.fori_loop(0, BW, tok, 0)

    def loop_body(i, carry):
        g0 = 2 * i
        g1 = 2 * i + 1
        @pl.when(lax.rem(g0 + 1, IBLK) == 0)
        def _():
            stage_idx(g0 + 1)
        @pl.when(i >= 1)
        def _():
            for c in scatter_copies(g0 - 1, va1, vc1, ss1):
                c.wait()
        issue_gathers(g0 + 1, va1, vb1, vc1, gs1)
        wait_gathers(g0, va0, vb0, vc0, gs0)
        compute(va0, vb0)
        for c in scatter_copies(g0, va0, vc0, ss0):
            c.start()
        @pl.when(i < (L // 2) - 1)
        def _():
            @pl.when(lax.rem(g1 + 1, IBLK) == 0)
            def _():
                stage_idx(g1 + 1)
            for c in scatter_copies(g0, va0, vc0, ss0):
                c.wait()
            issue_gathers(g1 + 1, va0, vb0, vc0, gs0)
        wait_gathers(g1, va1, vb1, vc1, gs1)
        compute(va1, vb1)
        for c in scatter_copies(g1, va1, vc1, ss1):
            c.start()
        return carry

    stage_idx(0)
    issue_gathers(0, va0, vb0, vc0, gs0)
    lax.fori_loop(0, L // 2, loop_body, 0)
    for c in scatter_copies(L - 2, va0, vc0, ss0):
        c.wait()
    for c in scatter_copies(L - 1, va1, vc1, ss1):
        c.wait()


_sc_gather = functools.partial(
    pl.kernel,
    out_type=jax.ShapeDtypeStruct((L, B, 2 * D), jnp.float32),
    mesh=plsc.VectorSubcoreMesh(core_axis_name="c", subcore_axis_name="s"),
    compiler_params=pltpu.CompilerParams(use_tc_tiling_on_sc=False),
    scratch_types=[
        pltpu.VMEM((IBLK, BW), jnp.int32),
        pltpu.VMEM((IBLK, BW), jnp.int32),
        pltpu.VMEM((BW, D), jnp.float32),
        pltpu.VMEM((BW, D), jnp.float32),
        pltpu.VMEM((BW, D), jnp.float32),
        pltpu.VMEM((BW, D), jnp.float32),
        pltpu.VMEM((BW, D), jnp.float32),
        pltpu.VMEM((BW, D), jnp.float32),
        pltpu.SemaphoreType.DMA,
        pltpu.SemaphoreType.DMA,
        pltpu.SemaphoreType.DMA,
        pltpu.SemaphoreType.DMA,
    ],
)(_sc_body)


_DT, _DR = D // 8, 8
_BT, _BC = B // 128, 128


def _relayout_body(mp_ref, m5_ref, p5_ref):
    x = mp_ref[0]
    for sl, dst in (((0, D), m5_ref), ((D, D), p5_ref)):
        xt = x[:, sl[0]:sl[0] + sl[1]].T
        dst[0] = xt.reshape(_DT, _DR, _BT, _BC).transpose(0, 2, 1, 3)


def _relayout(mp):
    m5, p5 = pl.pallas_call(
        _relayout_body,
        grid=(L,),
        in_specs=[
            pl.BlockSpec((1, B, 2 * D), lambda i: (i, 0, 0)),
        ],
        out_specs=[
            pl.BlockSpec((1, _DT, _BT, _DR, _BC), lambda i: (i, 0, 0, 0, 0)),
            pl.BlockSpec((1, _DT, _BT, _DR, _BC), lambda i: (i, 0, 0, 0, 0)),
        ],
        out_shape=[
            jax.ShapeDtypeStruct((L, _DT, _BT, _DR, _BC), jnp.float32),
            jax.ShapeDtypeStruct((L, _DT, _BT, _DR, _BC), jnp.float32),
        ],
    )(mp)

    def to3d(x5):
        return x5.transpose(2, 4, 0, 1, 3).reshape(B, L, D)

    return to3d(m5), to3d(p5)


def kernel(value_table, pos_table, W, b, inputs):
    vtt, ptt = _transform_tables(value_table, pos_table, W, b)
    vi = inputs[:, :, 0].T
    pi = inputs[:, :, 1].T
    mp = _sc_gather(vtt, ptt, pos_table, vi, pi)
    return _relayout(mp)


# speedup vs baseline: 2.2762x; 1.0276x over previous
import functools

import jax
import jax.numpy as jnp
from jax import lax
from jax.experimental import pallas as pl
from jax.experimental.pallas import tpu as pltpu
from jax.experimental.pallas import tpu_sc as plsc

WORD_VOCAB = 100000
D = 64
B = 4096
L = 200
NC, NS = 2, 16
NW = NC * NS
BW = B // NW
IBLK = 16


_R = 4000


def _transform_body(vt_ref, pt_ref, w_ref, b_ref, vtt_ref, ptt_ref):
    w = w_ref[...]
    wv = w[:, 0:64]
    wp = w[:, 64:128]
    dn = (((1,), (1,)), ((), ()))
    vtt_ref[...] = (
        lax.dot_general(vt_ref[...], wv, dn, precision=lax.Precision.HIGHEST)
        + b_ref[...]
    )
    ptt_ref[...] = lax.dot_general(
        pt_ref[...], wp, dn, precision=lax.Precision.HIGHEST
    )


def _transform_tables(value_table, pos_table, W, b):
    b2 = b.reshape(1, D)
    return pl.pallas_call(
        _transform_body,
        grid=(WORD_VOCAB // _R,),
        in_specs=[
            pl.BlockSpec((_R, D), lambda i: (i, 0)),
            pl.BlockSpec((_R, D), lambda i: (i, 0)),
            pl.BlockSpec((D, 2 * D), lambda i: (0, 0)),
            pl.BlockSpec((1, D), lambda i: (0, 0)),
        ],
        out_specs=[
            pl.BlockSpec((_R, D), lambda i: (i, 0)),
            pl.BlockSpec((_R, D), lambda i: (i, 0)),
        ],
        out_shape=[
            jax.ShapeDtypeStruct((WORD_VOCAB, D), jnp.float32),
            jax.ShapeDtypeStruct((WORD_VOCAB, D), jnp.float32),
        ],
    )(value_table, pos_table, W, b2)


def _sc_body(vtt, ptt, pt, idx4, mp,
             ivp, va0, vb0, vc0, va1, vb1, vc1,
             gs0, gs1, ss0, ss1):
    wid = lax.axis_index("s") * NC + lax.axis_index("c")
    b0 = wid * BW

    def stage_idx(g):
        pltpu.sync_copy(idx4.at[pl.ds(g, IBLK), wid], ivp)

    def gather_copies(g, va, vb, vc, sem):
        jj = lax.rem(g, IBLK)
        return [
            pltpu.make_async_copy(vtt.at[ivp.at[jj, 0]], va, sem),
            pltpu.make_async_copy(ptt.at[ivp.at[jj, 1]], vb, sem),
            pltpu.make_async_copy(pt.at[ivp.at[jj, 1]], vc, sem),
        ]

    def issue_gathers(g, va, vb, vc, sem):
        for c in gather_copies(g, va, vb, vc, sem):
            c.start()

    def wait_gathers(g, va, vb, vc, sem):
        for c in gather_copies(g, va, vb, vc, sem):
            c.wait()

    def scatter_copies(g, va, vc, sem):
        dst = mp.at[g, pl.ds(b0, BW)]
        return [
            pltpu.make_async_copy(va, dst.at[:, pl.ds(0, D)], sem),
            pltpu.make_async_copy(vc, dst.at[:, pl.ds(D, D)], sem),
        ]

    def compute(va, vb):
        def tok(t, carry):
            for dd in range(D // 16):
                sl = pl.ds(dd * 16, 16)
                va[t, sl] = jnp.maximum(va[t, sl] + vb[t, sl], 0.0)
            return carry
        lax.fori_loop(0, BW, tok, 0)

    def loop_body(i, carry):
        g0 = 2 * i
        g1 = 2 * i + 1
        @pl.when(lax.rem(g0 + 1, IBLK) == 0)
        def _():
            stage_idx(g0 + 1)
        @pl.when(i >= 1)
        def _():
            for c in scatter_copies(g0 - 1, va1, vc1, ss1):
                c.wait()
        issue_gathers(g0 + 1, va1, vb1, vc1, gs1)
        wait_gathers(g0, va0, vb0, vc0, gs0)
        compute(va0, vb0)
        for c in scatter_copies(g0, va0, vc0, ss0):
            c.start()
        @pl.when(i < (L // 2) - 1)
        def _():
            @pl.when(lax.rem(g1 + 1, IBLK) == 0)
            def _():
                stage_idx(g1 + 1)
            for c in scatter_copies(g0, va0, vc0, ss0):
                c.wait()
            issue_gathers(g1 + 1, va0, vb0, vc0, gs0)
        wait_gathers(g1, va1, vb1, vc1, gs1)
        compute(va1, vb1)
        for c in scatter_copies(g1, va1, vc1, ss1):
            c.start()
        return carry

    stage_idx(0)
    issue_gathers(0, va0, vb0, vc0, gs0)
    lax.fori_loop(0, L // 2, loop_body, 0)
    for c in scatter_copies(L - 2, va0, vc0, ss0):
        c.wait()
    for c in scatter_copies(L - 1, va1, vc1, ss1):
        c.wait()


_sc_gather = functools.partial(
    pl.kernel,
    out_type=jax.ShapeDtypeStruct((L, B, 2 * D), jnp.float32),
    mesh=plsc.VectorSubcoreMesh(core_axis_name="c", subcore_axis_name="s"),
    compiler_params=pltpu.CompilerParams(use_tc_tiling_on_sc=False),
    scratch_types=[
        pltpu.VMEM((IBLK, 2, BW), jnp.int32),
        pltpu.VMEM((BW, D), jnp.float32),
        pltpu.VMEM((BW, D), jnp.float32),
        pltpu.VMEM((BW, D), jnp.float32),
        pltpu.VMEM((BW, D), jnp.float32),
        pltpu.VMEM((BW, D), jnp.float32),
        pltpu.VMEM((BW, D), jnp.float32),
        pltpu.SemaphoreType.DMA,
        pltpu.SemaphoreType.DMA,
        pltpu.SemaphoreType.DMA,
        pltpu.SemaphoreType.DMA,
    ],
)(_sc_body)


_DT, _DR = D // 8, 8
_BT, _BC = B // 128, 128


def _relayout_body(mp_ref, m5_ref, p5_ref):
    x = mp_ref[0]
    for sl, dst in (((0, D), m5_ref), ((D, D), p5_ref)):
        xt = x[:, sl[0]:sl[0] + sl[1]].T
        dst[0] = xt.reshape(_DT, _DR, _BT, _BC).transpose(0, 2, 1, 3)


def _relayout(mp):
    m5, p5 = pl.pallas_call(
        _relayout_body,
        grid=(L,),
        in_specs=[
            pl.BlockSpec((1, B, 2 * D), lambda i: (i, 0, 0)),
        ],
        out_specs=[
            pl.BlockSpec((1, _DT, _BT, _DR, _BC), lambda i: (i, 0, 0, 0, 0)),
            pl.BlockSpec((1, _DT, _BT, _DR, _BC), lambda i: (i, 0, 0, 0, 0)),
        ],
        out_shape=[
            jax.ShapeDtypeStruct((L, _DT, _BT, _DR, _BC), jnp.float32),
            jax.ShapeDtypeStruct((L, _DT, _BT, _DR, _BC), jnp.float32),
        ],
    )(mp)

    def to3d(x5):
        return x5.transpose(2, 4, 0, 1, 3).reshape(B, L, D)

    return to3d(m5), to3d(p5)


def kernel(value_table, pos_table, W, b, inputs):
    vtt, ptt = _transform_tables(value_table, pos_table, W, b)
    idx4 = (inputs.transpose(1, 0, 2)
            .reshape(L, NW, BW, 2)
            .transpose(0, 1, 3, 2))
    mp = _sc_gather(vtt, ptt, pos_table, idx4)
    return _relayout(mp)


# speedup vs baseline: 2.3829x; 1.0469x over previous
import functools

import jax
import jax.numpy as jnp
from jax import lax
from jax.experimental import pallas as pl
from jax.experimental.pallas import tpu as pltpu
from jax.experimental.pallas import tpu_sc as plsc

WORD_VOCAB = 100000
D = 64
B = 4096
L = 200
NC, NS = 2, 16
NW = NC * NS
BW = B // NW
IBLK = 16


_R = 4000


def _transform_body(vt_ref, pt_ref, w_ref, b_ref, vtt_ref, ptt_ref):
    w = w_ref[...]
    wv = w[:, 0:64]
    wp = w[:, 64:128]
    dn = (((1,), (1,)), ((), ()))
    vtt_ref[...] = (
        lax.dot_general(vt_ref[...], wv, dn, precision=lax.Precision.HIGHEST)
        + b_ref[...]
    )
    ptt_ref[...] = lax.dot_general(
        pt_ref[...], wp, dn, precision=lax.Precision.HIGHEST
    )


def _transform_tables(value_table, pos_table, W, b):
    b2 = b.reshape(1, D)
    return pl.pallas_call(
        _transform_body,
        grid=(WORD_VOCAB // _R,),
        in_specs=[
            pl.BlockSpec((_R, D), lambda i: (i, 0)),
            pl.BlockSpec((_R, D), lambda i: (i, 0)),
            pl.BlockSpec((D, 2 * D), lambda i: (0, 0)),
            pl.BlockSpec((1, D), lambda i: (0, 0)),
        ],
        out_specs=[
            pl.BlockSpec((_R, D), lambda i: (i, 0)),
            pl.BlockSpec((_R, D), lambda i: (i, 0)),
        ],
        out_shape=[
            jax.ShapeDtypeStruct((WORD_VOCAB, D), jnp.float32),
            jax.ShapeDtypeStruct((WORD_VOCAB, D), jnp.float32),
        ],
    )(value_table, pos_table, W, b2)


def _sc_body(vtt, ptt, pt, idx4, mp,
             ivp, va0, vb0, vc0, va1, vb1, vc1,
             gs0, gs1, ss0, ss1):
    wid = lax.axis_index("s") * NC + lax.axis_index("c")
    b0 = wid * BW

    def stage_idx(g):
        pltpu.sync_copy(idx4.at[pl.ds(g, IBLK), wid], ivp)

    def gather_copies(g, va, vb, vc, sem):
        jj = lax.rem(g, IBLK)
        return [
            pltpu.make_async_copy(vtt.at[ivp.at[jj, 0]], va, sem),
            pltpu.make_async_copy(ptt.at[ivp.at[jj, 1]], vb, sem),
            pltpu.make_async_copy(pt.at[ivp.at[jj, 1]], vc, sem),
        ]

    def issue_gathers(g, va, vb, vc, sem):
        for c in gather_copies(g, va, vb, vc, sem):
            c.start()

    def wait_gathers(g, va, vb, vc, sem):
        for c in gather_copies(g, va, vb, vc, sem):
            c.wait()

    def scatter_copies(g, va, vc, sem):
        dst = mp.at[g, pl.ds(b0, BW)]
        return [
            pltpu.make_async_copy(va, dst.at[:, pl.ds(0, D)], sem),
            pltpu.make_async_copy(vc, dst.at[:, pl.ds(D, D)], sem),
        ]

    def compute(va, vb):
        def tok(t, carry):
            for dd in range(D // 16):
                sl = pl.ds(dd * 16, 16)
                va[t, sl] = jnp.maximum(va[t, sl] + vb[t, sl], 0.0)
            return carry
        lax.fori_loop(0, BW, tok, 0)

    def loop_body(i, carry):
        g0 = 2 * i
        g1 = 2 * i + 1
        @pl.when(lax.rem(g0 + 1, IBLK) == 0)
        def _():
            stage_idx(g0 + 1)
        @pl.when(i >= 1)
        def _():
            for c in scatter_copies(g0 - 1, va1, vc1, ss1):
                c.wait()
        issue_gathers(g0 + 1, va1, vb1, vc1, gs1)
        wait_gathers(g0, va0, vb0, vc0, gs0)
        compute(va0, vb0)
        for c in scatter_copies(g0, va0, vc0, ss0):
            c.start()
        @pl.when(i < (L // 2) - 1)
        def _():
            @pl.when(lax.rem(g1 + 1, IBLK) == 0)
            def _():
                stage_idx(g1 + 1)
            for c in scatter_copies(g0, va0, vc0, ss0):
                c.wait()
            issue_gathers(g1 + 1, va0, vb0, vc0, gs0)
        wait_gathers(g1, va1, vb1, vc1, gs1)
        compute(va1, vb1)
        for c in scatter_copies(g1, va1, vc1, ss1):
            c.start()
        return carry

    stage_idx(0)
    issue_gathers(0, va0, vb0, vc0, gs0)
    lax.fori_loop(0, L // 2, loop_body, 0)
    for c in scatter_copies(L - 2, va0, vc0, ss0):
        c.wait()
    for c in scatter_copies(L - 1, va1, vc1, ss1):
        c.wait()


_sc_gather = functools.partial(
    pl.kernel,
    out_type=jax.ShapeDtypeStruct((L, B, 2 * D), jnp.float32),
    mesh=plsc.VectorSubcoreMesh(core_axis_name="c", subcore_axis_name="s"),
    compiler_params=pltpu.CompilerParams(use_tc_tiling_on_sc=False),
    scratch_types=[
        pltpu.VMEM((IBLK, 2, BW), jnp.int32),
        pltpu.VMEM((BW, D), jnp.float32),
        pltpu.VMEM((BW, D), jnp.float32),
        pltpu.VMEM((BW, D), jnp.float32),
        pltpu.VMEM((BW, D), jnp.float32),
        pltpu.VMEM((BW, D), jnp.float32),
        pltpu.VMEM((BW, D), jnp.float32),
        pltpu.SemaphoreType.DMA,
        pltpu.SemaphoreType.DMA,
        pltpu.SemaphoreType.DMA,
        pltpu.SemaphoreType.DMA,
    ],
)(_sc_body)


_DT, _DR = D // 8, 8
_BT, _BC = B // 128, 128


def _relayout_body(mp_ref, m5_ref, p5_ref):
    for j in range(2):
        x = mp_ref[j]
        for sl, dst in (((0, D), m5_ref), ((D, D), p5_ref)):
            xt = x[:, sl[0]:sl[0] + sl[1]].T
            dst[j] = xt.reshape(_DT, _DR, _BT, _BC).transpose(0, 2, 1, 3)


def _relayout(mp):
    m5, p5 = pl.pallas_call(
        _relayout_body,
        grid=(L // 2,),
        in_specs=[
            pl.BlockSpec((2, B, 2 * D), lambda i: (i, 0, 0)),
        ],
        out_specs=[
            pl.BlockSpec((2, _DT, _BT, _DR, _BC), lambda i: (i, 0, 0, 0, 0)),
            pl.BlockSpec((2, _DT, _BT, _DR, _BC), lambda i: (i, 0, 0, 0, 0)),
        ],
        out_shape=[
            jax.ShapeDtypeStruct((L, _DT, _BT, _DR, _BC), jnp.float32),
            jax.ShapeDtypeStruct((L, _DT, _BT, _DR, _BC), jnp.float32),
        ],
    )(mp)

    def to3d(x5):
        return x5.transpose(2, 4, 0, 1, 3).reshape(B, L, D)

    return to3d(m5), to3d(p5)


def kernel(value_table, pos_table, W, b, inputs):
    vtt, ptt = _transform_tables(value_table, pos_table, W, b)
    idx4 = (inputs.transpose(1, 0, 2)
            .reshape(L, NW, BW, 2)
            .transpose(0, 1, 3, 2))
    mp = _sc_gather(vtt, ptt, pos_table, idx4)
    return _relayout(mp)
